# Initial kernel scaffold; baseline (speedup 1.0000x reference)
import jax
import jax.numpy as jnp
from jax.experimental import pallas as pl


def kernel(x, edge_index, batch, W1, b1, W2, b2, Wl, bl):
    raise NotImplementedError("write your pallas kernel here")


# speedup vs baseline: 10.8313x; 10.8313x over previous
import functools

import jax
import jax.numpy as jnp
from jax import lax
from jax.experimental import pallas as pl
from jax.experimental.pallas import tpu as pltpu
from jax.experimental.pallas import tpu_sc as plsc

F32 = jnp.float32
I32 = jnp.int32


def _sc_mesh():
    return plsc.VectorSubcoreMesh(core_axis_name="c", subcore_axis_name="s")


_SC_PARAMS = pltpu.CompilerParams(needs_layout_passes=False,
                                  use_tc_tiling_on_sc=False)


def _make_deg_kernel(E, NP):
    ER = E // 128
    half_rows = ER // 2
    slc = NP // 16

    @functools.partial(
        pl.kernel,
        out_type=jax.ShapeDtypeStruct((2 * NP,), F32),
        mesh=_sc_mesh(),
        compiler_params=_SC_PARAMS,
        scratch_types=[
            pltpu.VMEM((128,), I32),
            pltpu.VMEM((128,), F32),
            pltpu.VMEM((slc,), F32),
            pltpu.VMEM_SHARED((NP,), F32),
        ],
    )
    def k(dst_hbm, zeros_hbm, out_hbm, dbuf, obuf, bbuf, acc):
        c = lax.axis_index("c")
        s = lax.axis_index("s")
        for j in range(8):
            obuf[pl.ds(j * 16, 16)] = jnp.full((16,), 1.0, F32)
        pltpu.sync_copy(zeros_hbm, acc.at[pl.ds(s * slc, slc)])
        plsc.subcore_barrier()

        @pl.loop(c * half_rows + s, (c + 1) * half_rows, step=16)
        def _(r):
            pltpu.sync_copy(dst_hbm.at[pl.ds(r * 128, 128)], dbuf)
            pltpu.sync_copy(obuf, acc.at[dbuf], add=True)

        plsc.subcore_barrier()
        pltpu.sync_copy(acc.at[pl.ds(s * slc, slc)], bbuf)
        pltpu.sync_copy(bbuf, out_hbm.at[pl.ds(c * NP + s * slc, slc)])

    return k


def _make_scalar_prop_kernel(E, NP):
    ER = E // 128
    half_rows = ER // 2
    slc = NP // 16

    @functools.partial(
        pl.kernel,
        out_type=jax.ShapeDtypeStruct((2 * NP,), F32),
        mesh=_sc_mesh(),
        compiler_params=_SC_PARAMS,
        scratch_types=[
            pltpu.VMEM((128,), I32),
            pltpu.VMEM((128,), I32),
            pltpu.VMEM((128,), F32),
            pltpu.VMEM((slc,), F32),
            pltpu.VMEM((NP,), F32),
            pltpu.VMEM_SHARED((NP,), F32),
        ],
    )
    def k(src_hbm, dst_hbm, u0_hbm, zeros_hbm, out_hbm,
          sbuf, dbuf, vbuf, bbuf, u0v, acc):
        c = lax.axis_index("c")
        s = lax.axis_index("s")
        pltpu.sync_copy(zeros_hbm, acc.at[pl.ds(s * slc, slc)])
        pltpu.sync_copy(u0_hbm, u0v)
        plsc.subcore_barrier()

        @pl.loop(c * half_rows + s, (c + 1) * half_rows, step=16)
        def _(r):
            pltpu.sync_copy(src_hbm.at[pl.ds(r * 128, 128)], sbuf)
            pltpu.sync_copy(dst_hbm.at[pl.ds(r * 128, 128)], dbuf)
            for j in range(8):
                idx = sbuf[pl.ds(j * 16, 16)]
                vbuf[pl.ds(j * 16, 16)] = plsc.load_gather(u0v, [idx])
            pltpu.sync_copy(vbuf, acc.at[dbuf], add=True)

        plsc.subcore_barrier()
        pltpu.sync_copy(acc.at[pl.ds(s * slc, slc)], bbuf)
        pltpu.sync_copy(bbuf, out_hbm.at[pl.ds(c * NP + s * slc, slc)])

    return k


def _make_row_prop_kernel(E, NP, N, D):
    ER = E // 128
    HALF = N // 2
    zrows = 3128
    arows = 16 * zrows
    w0 = 3080

    @functools.partial(
        pl.kernel,
        out_type=(jax.ShapeDtypeStruct((NP, D), F32),
                  jax.ShapeDtypeStruct((NP, D), F32)),
        mesh=_sc_mesh(),
        compiler_params=_SC_PARAMS,
        scratch_types=[
            pltpu.VMEM((128,), I32),
            pltpu.VMEM((128,), I32),
            pltpu.VMEM((128,), I32),
            pltpu.VMEM((128, D), F32),
            pltpu.VMEM((zrows, D), F32),
            pltpu.VMEM_SHARED((arows, D), F32),
            pltpu.SemaphoreType.DMA,
        ],
    )
    def k(src_hbm, dst_hbm, u1a_hbm, u1b_hbm, zeros_hbm, ta_hbm, tb_hbm,
          sbuf, dbuf, lbuf, rows, bbuf, acc, sem):
        c = lax.axis_index("c")
        s = lax.axis_index("s")
        base = c * HALF
        garb = HALF + s

        for u_hbm, o_hbm in ((u1a_hbm, ta_hbm), (u1b_hbm, tb_hbm)):
            pltpu.sync_copy(zeros_hbm, acc.at[pl.ds(s * zrows, zrows)])
            plsc.subcore_barrier()

            @pl.loop(s, ER, step=16)
            def _(r):
                pltpu.sync_copy(src_hbm.at[pl.ds(r * 128, 128)], sbuf)
                pltpu.sync_copy(dst_hbm.at[pl.ds(r * 128, 128)], dbuf)
                cp = pltpu.async_copy(u_hbm.at[sbuf], rows, sem)
                for j in range(8):
                    d = dbuf[pl.ds(j * 16, 16)]
                    loc = d - base
                    ok = (loc >= 0) & (loc < HALF)
                    lbuf[pl.ds(j * 16, 16)] = jnp.where(ok, loc, garb)
                cp.wait()
                pltpu.sync_copy(rows, acc.at[lbuf], add=True)

            plsc.subcore_barrier()
            pltpu.sync_copy(acc.at[pl.ds(s * zrows, w0)],
                            bbuf.at[pl.ds(0, w0)])
            pltpu.sync_copy(bbuf.at[pl.ds(0, w0)],
                            o_hbm.at[pl.ds(base + s * zrows, w0)])

            @pl.when(s < 15)
            def _():
                pltpu.sync_copy(acc.at[pl.ds(s * zrows + w0, zrows - w0)],
                                bbuf.at[pl.ds(w0, zrows - w0)])
                pltpu.sync_copy(bbuf.at[pl.ds(w0, zrows - w0)],
                                o_hbm.at[pl.ds(base + s * zrows + w0,
                                               zrows - w0)])

            plsc.subcore_barrier()

    return k


def _dinv_u0_body(d0, d1, x, dinv_o, u0_o):
    deg = d0[...] + d1[...] + 1.0
    dinv = lax.rsqrt(deg)
    dinv_o[...] = dinv
    u0_o[...] = dinv * x[...]


def _layer1_body(t0, t1, u0, dinv, w1, b1, ua_o, ub_o):
    p = dinv[...] * (t0[...] + t1[...] + u0[...])
    h1 = jnp.maximum(p * w1[...] + b1[...], 0.0)
    u1 = dinv[...] * h1
    ua_o[...] = u1[:, :16]
    ub_o[...] = u1[:, 16:]


def _layer2_pool_body(ta, tb, ua, ub, dinv, bat, w2, b2, wl, bl,
                      out_o, sums, cnts):
    i = pl.program_id(0)
    q32 = jnp.concatenate([ta[...] + ua[...], tb[...] + ub[...]], axis=1)
    q = dinv[...] * q32
    h2 = jnp.maximum(
        jnp.dot(q, w2[...], preferred_element_type=F32) + b2[...], 0.0)
    v = jnp.dot(h2, wl[...], preferred_element_type=F32)
    b = bat[...]
    valid = b < 256
    v = jnp.where(valid, v, 0.0)
    gids = lax.broadcasted_iota(I32, (b.shape[0], 256), 1)
    onehot = (b == gids).astype(F32)
    dn = (((0,), (0,)), ((), ()))
    s_c = lax.dot_general(onehot, v, dn, preferred_element_type=F32)
    ones = jnp.where(valid, 1.0, 0.0)
    c_c = lax.dot_general(onehot, ones, dn, preferred_element_type=F32)

    @pl.when(i == 0)
    def _():
        sums[...] = s_c
        cnts[...] = c_c

    @pl.when(i > 0)
    def _():
        sums[...] += s_c
        cnts[...] += c_c

    @pl.when(i == pl.num_programs(0) - 1)
    def _():
        out_o[...] = sums[...] / jnp.maximum(cnts[...], 1.0) + bl[...]


def kernel(x, edge_index, batch, W1, b1, W2, b2, Wl, bl):
    N = x.shape[0]
    E = edge_index.shape[1]
    G = 256
    NP = 100352
    BLK = 2048
    NB = NP // BLK
    D = 16

    src = edge_index[0]
    dst = edge_index[1]
    xp = jnp.pad(x[:, 0], (0, NP - N)).reshape(784, 128)
    batp = jnp.pad(batch, (0, NP - N), constant_values=G).reshape(NP, 1)
    zeros1 = jnp.zeros((NP // 16,), F32)
    zeros2 = jnp.zeros((3128, D), F32)

    degp = _make_deg_kernel(E, NP)(dst, zeros1)

    dinv2, u02 = pl.pallas_call(
        _dinv_u0_body,
        out_shape=[jax.ShapeDtypeStruct((784, 128), F32)] * 2,
    )(degp[:NP].reshape(784, 128), degp[NP:].reshape(784, 128), xp)
    dinv_c = dinv2.reshape(NP, 1)
    u0_f = u02.reshape(NP)

    tp = _make_scalar_prop_kernel(E, NP)(src, dst, u0_f, zeros1)

    col = pl.BlockSpec((BLK, 1), lambda i: (i, 0))
    half = pl.BlockSpec((BLK, D), lambda i: (i, 0))
    u1a, u1b = pl.pallas_call(
        _layer1_body,
        grid=(NB,),
        in_specs=[col, col, col, col,
                  pl.BlockSpec((1, 32), lambda i: (0, 0)),
                  pl.BlockSpec((1, 32), lambda i: (0, 0))],
        out_specs=[half, half],
        out_shape=[jax.ShapeDtypeStruct((NP, D), F32)] * 2,
    )(tp[:NP].reshape(NP, 1), tp[NP:].reshape(NP, 1), u0_f.reshape(NP, 1),
      dinv_c, W1.reshape(1, 32), b1.reshape(1, 32))

    t2a, t2b = _make_row_prop_kernel(E, NP, N, D)(src, dst, u1a, u1b, zeros2)

    out = pl.pallas_call(
        _layer2_pool_body,
        grid=(NB,),
        in_specs=[half, half, half, half, col,
                  pl.BlockSpec((BLK, 1), lambda i: (i, 0)),
                  pl.BlockSpec((32, 64), lambda i: (0, 0)),
                  pl.BlockSpec((1, 64), lambda i: (0, 0)),
                  pl.BlockSpec((64, 1), lambda i: (0, 0)),
                  pl.BlockSpec((1, 1), lambda i: (0, 0))],
        out_specs=pl.BlockSpec((G, 1), lambda i: (0, 0)),
        out_shape=jax.ShapeDtypeStruct((G, 1), F32),
        scratch_shapes=[pltpu.VMEM((G, 1), F32), pltpu.VMEM((G, 1), F32)],
    )(t2a, t2b, u1a, u1b, dinv_c, batp, W2, b2.reshape(1, 64), Wl,
      bl.reshape(1, 1))
    return out


# speedup vs baseline: 48.5221x; 4.4798x over previous
import functools

import jax
import jax.numpy as jnp
from jax import lax
from jax.experimental import pallas as pl
from jax.experimental.pallas import tpu as pltpu
from jax.experimental.pallas import tpu_sc as plsc

F32 = jnp.float32
I32 = jnp.int32

EPG = 512
KSUB = EPG // 128


def _sc_mesh():
    return plsc.VectorSubcoreMesh(core_axis_name="c", subcore_axis_name="s")


_SC_PARAMS = pltpu.CompilerParams(needs_layout_passes=False,
                                  use_tc_tiling_on_sc=False)


def _make_deg_scalar_kernel(E, NP, gather):
    GROUPS = E // EPG
    PER_C = GROUPS // 2
    NIT = 98
    slc = NP // 16

    scratch = [
        pltpu.VMEM((EPG,), I32),
        pltpu.VMEM((KSUB, 128), I32),
        pltpu.VMEM((KSUB, 128), I32),
        pltpu.VMEM((KSUB, 128), F32),
        pltpu.VMEM((EPG,), I32),
        pltpu.VMEM((KSUB, 128), I32),
        pltpu.VMEM((KSUB, 128), I32),
        pltpu.VMEM((KSUB, 128), F32),
        pltpu.VMEM((slc,), F32),
        pltpu.VMEM((NP,), F32),
        pltpu.VMEM_SHARED((NP,), F32),
        pltpu.SemaphoreType.DMA,
        pltpu.SemaphoreType.DMA,
        pltpu.SemaphoreType.DMA,
        pltpu.SemaphoreType.DMA,
    ]

    @functools.partial(
        pl.kernel,
        out_type=jax.ShapeDtypeStruct((2 * NP,), F32),
        mesh=_sc_mesh(),
        compiler_params=_SC_PARAMS,
        scratch_types=scratch,
    )
    def k(src_hbm, dst2_hbm, u0_hbm, zeros_hbm, out_hbm,
          sb0, db0, ib0, vb0, sb1, db1, ib1, vb1, bbuf, u0v, acc,
          es0, es1, ss0, ss1):
        c = lax.axis_index("c")
        s = lax.axis_index("s")
        sets = ((sb0, db0, ib0, vb0, es0, ss0),
                (sb1, db1, ib1, vb1, es1, ss1))
        start = c * PER_C
        cnt = PER_C + c
        pltpu.sync_copy(zeros_hbm, acc.at[pl.ds(s * slc, slc)])
        if gather:
            pltpu.sync_copy(u0_hbm, u0v)
        plsc.subcore_barrier()

        def g_of(ii):
            return start + s + 16 * ii

        def valid(g):
            return (g >= start) & (g < start + cnt)

        def fire_edges(g, st):
            sb, db = st[0], st[1]
            if gather:
                pltpu.async_copy(src_hbm.at[pl.ds(g * EPG, EPG)], sb, st[4])
            pltpu.async_copy(dst2_hbm.at[pl.ds(g * KSUB, KSUB)], db, st[4])

        def wait_edges(g, st):
            sb, db = st[0], st[1]
            if gather:
                pltpu.make_async_copy(
                    src_hbm.at[pl.ds(g * EPG, EPG)], sb, st[4]).wait()
            pltpu.make_async_copy(
                dst2_hbm.at[pl.ds(g * KSUB, KSUB)], db, st[4]).wait()

        def drain_scatters(st):
            for kk in range(KSUB):
                pltpu.make_async_copy(
                    st[3].at[kk], acc.at[st[2].at[kk]], st[5]).wait()

        def do_slot(ii, b):
            st = sets[b]
            sb, db, ib, vb, es, ss = st
            g = g_of(ii)

            @pl.when(valid(g))
            def _():
                wait_edges(g, st)

                @pl.when(ii >= 2)
                def _():
                    drain_scatters(st)

                for kk in range(KSUB):
                    for j in range(8):
                        sl = pl.ds(j * 16, 16)
                        ib[kk, sl] = db[kk, sl]
                        if gather:
                            idx = sb[pl.ds(kk * 128 + j * 16, 16)]
                            vb[kk, sl] = plsc.load_gather(u0v, [idx])
                        else:
                            vb[kk, sl] = jnp.full((16,), 1.0, F32)
                for kk in range(KSUB):
                    pltpu.async_copy(vb.at[kk], acc.at[ib.at[kk]], ss,
                                     add=True)

                @pl.when(valid(g_of(ii + 2)))
                def _():
                    fire_edges(g_of(ii + 2), st)

        for b in (0, 1):
            @pl.when(valid(g_of(b)))
            def _(b=b):
                fire_edges(g_of(b), sets[b])

        @pl.loop(0, NIT, step=2)
        def _(i):
            do_slot(i, 0)
            do_slot(i + 1, 1)

        for ii in (NIT - 3, NIT - 2, NIT - 1):
            @pl.when(valid(g_of(ii)) & ~valid(g_of(ii + 2)))
            def _(ii=ii):
                drain_scatters(sets[ii % 2])

        plsc.subcore_barrier()
        pltpu.sync_copy(acc.at[pl.ds(s * slc, slc)], bbuf)
        pltpu.sync_copy(bbuf, out_hbm.at[pl.ds(c * NP + s * slc, slc)])

    return k


def _make_row_prop_kernel(E, NP, N, D):
    GROUPS = E // EPG
    NIT = 198
    HALF = N // 2
    zrows = 3128
    arows = 16 * zrows
    w0 = 3080

    scratch = []
    for _ in range(3):
        scratch += [
            pltpu.VMEM((EPG,), I32),
            pltpu.VMEM((EPG,), I32),
            pltpu.VMEM((KSUB, 128), I32),
            pltpu.VMEM((EPG, D), F32),
            pltpu.SemaphoreType.DMA,
            pltpu.SemaphoreType.DMA,
            pltpu.SemaphoreType.DMA,
        ]
    scratch += [
        pltpu.VMEM((zrows, D), F32),
        pltpu.VMEM_SHARED((arows, D), F32),
    ]

    @functools.partial(
        pl.kernel,
        out_type=(jax.ShapeDtypeStruct((NP, D), F32),
                  jax.ShapeDtypeStruct((NP, D), F32)),
        mesh=_sc_mesh(),
        compiler_params=_SC_PARAMS,
        scratch_types=scratch,
    )
    def k(src_hbm, dst_hbm, u1a_hbm, u1b_hbm, zeros_hbm, ta_hbm, tb_hbm,
          *refs):
        sets = tuple(refs[7 * m:7 * m + 7] for m in range(3))
        bbuf, acc = refs[21], refs[22]
        c = lax.axis_index("c")
        s = lax.axis_index("s")
        base = c * HALF
        garb = HALF + s

        def g_of(ii):
            return s + 16 * ii

        def valid(g):
            return (g >= 0) & (g < GROUPS)

        def fire_edges(g, st):
            pltpu.async_copy(src_hbm.at[pl.ds(g * EPG, EPG)], st[0], st[4])
            pltpu.async_copy(dst_hbm.at[pl.ds(g * EPG, EPG)], st[1], st[4])

        def wait_edges(g, st):
            pltpu.make_async_copy(
                src_hbm.at[pl.ds(g * EPG, EPG)], st[0], st[4]).wait()
            pltpu.make_async_copy(
                dst_hbm.at[pl.ds(g * EPG, EPG)], st[1], st[4]).wait()

        for u_hbm, o_hbm in ((u1a_hbm, ta_hbm), (u1b_hbm, tb_hbm)):
            pltpu.sync_copy(zeros_hbm, acc.at[pl.ds(s * zrows, zrows)])
            plsc.subcore_barrier()

            def fire_gathers(st):
                for kk in range(KSUB):
                    pltpu.async_copy(u_hbm.at[st[0].at[pl.ds(kk * 128, 128)]],
                                     st[3].at[pl.ds(kk * 128, 128)], st[5])

            def drain_gathers(st):
                for kk in range(KSUB):
                    pltpu.make_async_copy(
                        u_hbm.at[st[0].at[pl.ds(kk * 128, 128)]],
                        st[3].at[pl.ds(kk * 128, 128)], st[5]).wait()

            def fire_scatters(st):
                for kk in range(KSUB):
                    pltpu.async_copy(st[3].at[pl.ds(kk * 128, 128)],
                                     acc.at[st[2].at[kk]], st[6], add=True)

            def drain_scatters(st):
                for kk in range(KSUB):
                    pltpu.make_async_copy(
                        st[3].at[pl.ds(kk * 128, 128)],
                        acc.at[st[2].at[kk]], st[6]).wait()

            def do_slot(ii, m):
                st = sets[m]
                stp = sets[(m + 2) % 3]
                db, lb = st[1], st[2]
                g = g_of(ii)
                gp = g_of(ii - 1)

                @pl.when(valid(g))
                def _():
                    wait_edges(g, st)

                    @pl.when(ii >= 3)
                    def _():
                        drain_scatters(st)

                    fire_gathers(st)
                    for kk in range(KSUB):
                        for j in range(8):
                            d = db[pl.ds(kk * 128 + j * 16, 16)]
                            loc = d - base
                            ok = (loc >= 0) & (loc < HALF)
                            lb[kk, pl.ds(j * 16, 16)] = \
                                jnp.where(ok, loc, garb)

                @pl.when(valid(gp))
                def _():
                    drain_gathers(stp)
                    fire_scatters(stp)

                    @pl.when(valid(g_of(ii + 2)))
                    def _():
                        fire_edges(g_of(ii + 2), stp)

            for m in (0, 1, 2):
                @pl.when(valid(g_of(m)))
                def _(m=m):
                    fire_edges(g_of(m), sets[m])

            @pl.loop(0, NIT, step=3)
            def _(i):
                do_slot(i, 0)
                do_slot(i + 1, 1)
                do_slot(i + 2, 2)

            for ii in range(NIT - 6, NIT):
                @pl.when(valid(g_of(ii)) & ~valid(g_of(ii + 3)))
                def _(ii=ii):
                    drain_scatters(sets[ii % 3])

            plsc.subcore_barrier()
            pltpu.sync_copy(acc.at[pl.ds(s * zrows, w0)],
                            bbuf.at[pl.ds(0, w0)])
            pltpu.sync_copy(bbuf.at[pl.ds(0, w0)],
                            o_hbm.at[pl.ds(base + s * zrows, w0)])

            @pl.when(s < 15)
            def _():
                pltpu.sync_copy(acc.at[pl.ds(s * zrows + w0, zrows - w0)],
                                bbuf.at[pl.ds(w0, zrows - w0)])
                pltpu.sync_copy(bbuf.at[pl.ds(w0, zrows - w0)],
                                o_hbm.at[pl.ds(base + s * zrows + w0,
                                               zrows - w0)])

            plsc.subcore_barrier()

    return k


def _dinv_u0_body(d0, d1, x, dinv_o, u0_o):
    deg = d0[...] + d1[...] + 1.0
    dinv = lax.rsqrt(deg)
    dinv_o[...] = dinv
    u0_o[...] = dinv * x[...]


def _layer1_body(t0, t1, u0, dinv, w1, b1, ua_o, ub_o):
    p = dinv[...] * (t0[...] + t1[...] + u0[...])
    h1 = jnp.maximum(p * w1[...] + b1[...], 0.0)
    u1 = dinv[...] * h1
    ua_o[...] = u1[:, :16]
    ub_o[...] = u1[:, 16:]


def _layer2_pool_body(ta, tb, ua, ub, dinv, bat, w2, b2, wl, bl,
                      out_o, sums, cnts):
    i = pl.program_id(0)
    q32 = jnp.concatenate([ta[...] + ua[...], tb[...] + ub[...]], axis=1)
    q = dinv[...] * q32
    h2 = jnp.maximum(
        jnp.dot(q, w2[...], preferred_element_type=F32) + b2[...], 0.0)
    v = jnp.dot(h2, wl[...], preferred_element_type=F32)
    b = bat[...]
    valid = b < 256
    v = jnp.where(valid, v, 0.0)
    gids = lax.broadcasted_iota(I32, (b.shape[0], 256), 1)
    onehot = (b == gids).astype(F32)
    dn = (((0,), (0,)), ((), ()))
    s_c = lax.dot_general(onehot, v, dn, preferred_element_type=F32)
    ones = jnp.where(valid, 1.0, 0.0)
    c_c = lax.dot_general(onehot, ones, dn, preferred_element_type=F32)

    @pl.when(i == 0)
    def _():
        sums[...] = s_c
        cnts[...] = c_c

    @pl.when(i > 0)
    def _():
        sums[...] += s_c
        cnts[...] += c_c

    @pl.when(i == pl.num_programs(0) - 1)
    def _():
        out_o[...] = sums[...] / jnp.maximum(cnts[...], 1.0) + bl[...]


def kernel(x, edge_index, batch, W1, b1, W2, b2, Wl, bl):
    N = x.shape[0]
    E = edge_index.shape[1]
    G = 256
    NP = 100352
    BLK = 2048
    NB = NP // BLK
    D = 16

    src = edge_index[0]
    dst = edge_index[1]
    dst2 = dst.reshape(E // 128, 128)
    xp = jnp.pad(x[:, 0], (0, NP - N)).reshape(784, 128)
    batp = jnp.pad(batch, (0, NP - N), constant_values=G).reshape(NP, 1)
    zeros1 = jnp.zeros((NP // 16,), F32)
    zeros2 = jnp.zeros((3128, D), F32)

    degp = _make_deg_scalar_kernel(E, NP, gather=False)(
        src, dst2, zeros1, zeros1)

    dinv2, u02 = pl.pallas_call(
        _dinv_u0_body,
        out_shape=[jax.ShapeDtypeStruct((784, 128), F32)] * 2,
    )(degp[:NP].reshape(784, 128), degp[NP:].reshape(784, 128), xp)
    dinv_c = dinv2.reshape(NP, 1)
    u0_f = u02.reshape(NP)

    tp = _make_deg_scalar_kernel(E, NP, gather=True)(
        src, dst2, u0_f, zeros1)

    col = pl.BlockSpec((BLK, 1), lambda i: (i, 0))
    half = pl.BlockSpec((BLK, D), lambda i: (i, 0))
    u1a, u1b = pl.pallas_call(
        _layer1_body,
        grid=(NB,),
        in_specs=[col, col, col, col,
                  pl.BlockSpec((1, 32), lambda i: (0, 0)),
                  pl.BlockSpec((1, 32), lambda i: (0, 0))],
        out_specs=[half, half],
        out_shape=[jax.ShapeDtypeStruct((NP, D), F32)] * 2,
    )(tp[:NP].reshape(NP, 1), tp[NP:].reshape(NP, 1), u0_f.reshape(NP, 1),
      dinv_c, W1.reshape(1, 32), b1.reshape(1, 32))

    t2a, t2b = _make_row_prop_kernel(E, NP, N, D)(src, dst, u1a, u1b, zeros2)

    out = pl.pallas_call(
        _layer2_pool_body,
        grid=(NB,),
        in_specs=[half, half, half, half, col,
                  pl.BlockSpec((BLK, 1), lambda i: (i, 0)),
                  pl.BlockSpec((32, 64), lambda i: (0, 0)),
                  pl.BlockSpec((1, 64), lambda i: (0, 0)),
                  pl.BlockSpec((64, 1), lambda i: (0, 0)),
                  pl.BlockSpec((1, 1), lambda i: (0, 0))],
        out_specs=pl.BlockSpec((G, 1), lambda i: (0, 0)),
        out_shape=jax.ShapeDtypeStruct((G, 1), F32),
        scratch_shapes=[pltpu.VMEM((G, 1), F32), pltpu.VMEM((G, 1), F32)],
    )(t2a, t2b, u1a, u1b, dinv_c, batp, W2, b2.reshape(1, 64), Wl,
      bl.reshape(1, 1))
    return out


# speedup vs baseline: 69.1479x; 1.4251x over previous
import functools

import jax
import jax.numpy as jnp
from jax import lax
from jax.experimental import pallas as pl
from jax.experimental.pallas import tpu as pltpu
from jax.experimental.pallas import tpu_sc as plsc

F32 = jnp.float32
I32 = jnp.int32

EPG = 512
KSUB = EPG // 128


def _sc_mesh():
    return plsc.VectorSubcoreMesh(core_axis_name="c", subcore_axis_name="s")


_SC_PARAMS = pltpu.CompilerParams(needs_layout_passes=False,
                                  use_tc_tiling_on_sc=False)


def _splat16(v):
    return jnp.full((16,), v, I32)


def _make_deg_scalar_kernel(E, NP, gather):
    GROUPS = E // EPG
    PER_C = GROUPS // 2
    NIT = 98
    slc = NP // 16

    scratch = []
    for _ in range(2):
        scratch += [
            pltpu.VMEM((KSUB, 128), I32),
            pltpu.VMEM((KSUB, 128), I32),
            pltpu.VMEM((KSUB, 128), I32),
            pltpu.VMEM((KSUB, 128), F32),
            pltpu.SemaphoreType.DMA,
            pltpu.SemaphoreType.DMA,
        ]
    scratch += [
        pltpu.VMEM((slc,), F32),
        pltpu.VMEM((NP,), F32),
        pltpu.VMEM_SHARED((NP,), F32),
    ]

    @functools.partial(
        pl.kernel,
        out_type=jax.ShapeDtypeStruct((2 * NP,), F32),
        mesh=_sc_mesh(),
        compiler_params=_SC_PARAMS,
        scratch_types=scratch,
    )
    def k(ei_hbm, u0_hbm, zeros_hbm, out_hbm, *refs):
        sets = tuple(refs[6 * m:6 * m + 6] for m in range(2))
        bbuf, u0v, acc = refs[12], refs[13], refs[14]
        c = lax.axis_index("c")
        s = lax.axis_index("s")
        start = c * PER_C
        cnt = PER_C + c
        pltpu.sync_copy(zeros_hbm, acc.at[pl.ds(s * slc, slc)])
        if gather:
            pltpu.sync_copy(u0_hbm, u0v)
        plsc.subcore_barrier()

        def g_of(ii):
            return start + s + 16 * ii

        def valid(g):
            return g < start + cnt

        def fire_edges(g, st):
            if gather:
                pltpu.async_copy(ei_hbm.at[0, pl.ds(g * KSUB, KSUB)],
                                 st[0], st[4])
            pltpu.async_copy(ei_hbm.at[1, pl.ds(g * KSUB, KSUB)],
                             st[1], st[4])

        def wait_edges(g, st):
            if gather:
                pltpu.make_async_copy(ei_hbm.at[0, pl.ds(g * KSUB, KSUB)],
                                      st[0], st[4]).wait()
            pltpu.make_async_copy(ei_hbm.at[1, pl.ds(g * KSUB, KSUB)],
                                  st[1], st[4]).wait()

        def drain_scatters(st):
            for kk in range(KSUB):
                pltpu.make_async_copy(
                    st[3].at[kk], acc.at[st[2].at[kk]], st[5]).wait()

        def do_slot(ii, b):
            st = sets[b]
            sb, db, ib, vb, es, ss = st
            g = g_of(ii)

            @pl.when(valid(g))
            def _():
                wait_edges(g, st)

                @pl.when(ii >= 2)
                def _():
                    drain_scatters(st)

                for kk in range(KSUB):
                    for j in range(8):
                        sl = pl.ds(j * 16, 16)
                        ib[kk, sl] = db[kk, sl]
                        if gather:
                            vb[kk, sl] = plsc.load_gather(u0v, [sb[kk, sl]])
                        else:
                            vb[kk, sl] = jnp.full((16,), 1.0, F32)
                for kk in range(KSUB):
                    pltpu.async_copy(vb.at[kk], acc.at[ib.at[kk]], ss,
                                     add=True)

                @pl.when(valid(g_of(ii + 2)))
                def _():
                    fire_edges(g_of(ii + 2), st)

        for b in (0, 1):
            @pl.when(valid(g_of(b)))
            def _(b=b):
                fire_edges(g_of(b), sets[b])

        @pl.loop(0, NIT, step=2)
        def _(i):
            do_slot(i, 0)
            do_slot(i + 1, 1)

        for ii in (NIT - 3, NIT - 2, NIT - 1):
            @pl.when(valid(g_of(ii)) & ~valid(g_of(ii + 2)))
            def _(ii=ii):
                drain_scatters(sets[ii % 2])

        plsc.subcore_barrier()
        pltpu.sync_copy(acc.at[pl.ds(s * slc, slc)], bbuf)
        pltpu.sync_copy(bbuf, out_hbm.at[pl.ds(c * NP + s * slc, slc)])

    return k


def _make_u1_kernel(NP):
    CH = 768
    TAIL = 128

    @functools.partial(
        pl.kernel,
        out_type=(jax.ShapeDtypeStruct((NP, 16), F32),
                  jax.ShapeDtypeStruct((NP, 16), F32),
                  jax.ShapeDtypeStruct((NP, 16), F32)),
        mesh=_sc_mesh(),
        compiler_params=_SC_PARAMS,
        scratch_types=[
            pltpu.VMEM((CH,), F32),
            pltpu.VMEM((CH,), F32),
            pltpu.VMEM((CH,), F32),
            pltpu.VMEM((CH,), F32),
            pltpu.VMEM((CH,), F32),
            pltpu.VMEM((CH, 16), F32),
            pltpu.VMEM((CH, 16), F32),
            pltpu.VMEM((CH, 16), F32),
            pltpu.VMEM((32,), F32),
            pltpu.VMEM((32,), F32),
        ],
    )
    def k(tp_hbm, u0_hbm, dv_hbm, w1_hbm, b1_hbm, ua_hbm, ub_hbm, db_hbm,
          t0b, t1b, u0b, dvs, pb, oa, ob, od, w1v, b1v):
        c = lax.axis_index("c")
        s = lax.axis_index("s")
        pltpu.sync_copy(w1_hbm, w1v)
        pltpu.sync_copy(b1_hbm, b1v)
        iota16 = lax.iota(I32, 16)

        def run_chunk(n0, ch):
            ng = ch // 16
            pltpu.sync_copy(tp_hbm.at[pl.ds(n0, ch)], t0b.at[pl.ds(0, ch)])
            pltpu.sync_copy(tp_hbm.at[pl.ds(NP + n0, ch)],
                            t1b.at[pl.ds(0, ch)])
            pltpu.sync_copy(u0_hbm.at[pl.ds(n0, ch)], u0b.at[pl.ds(0, ch)])
            pltpu.sync_copy(dv_hbm.at[pl.ds(n0, ch)], dvs.at[pl.ds(0, ch)])

            @pl.loop(0, ng)
            def _(g):
                sl = pl.ds(g * 16, 16)
                dv = dvs[sl]
                pb[sl] = dv * (t0b[sl] + t1b[sl] + u0b[sl])
                idxn = g * 16 + iota16
                for fc in range(16):
                    plsc.store_scatter(od, [idxn, _splat16(fc)], dv)

            for fc in range(16):
                wfa = plsc.load_gather(w1v, [_splat16(fc)])
                bfa = plsc.load_gather(b1v, [_splat16(fc)])
                wfb = plsc.load_gather(w1v, [_splat16(fc + 16)])
                bfb = plsc.load_gather(b1v, [_splat16(fc + 16)])

                @pl.loop(0, ng)
                def _(g):
                    sl = pl.ds(g * 16, 16)
                    p = pb[sl]
                    dv = dvs[sl]
                    idxn = g * 16 + iota16
                    ha = jnp.maximum(p * wfa + bfa, 0.0) * dv
                    plsc.store_scatter(oa, [idxn, _splat16(fc)], ha)
                    hb = jnp.maximum(p * wfb + bfb, 0.0) * dv
                    plsc.store_scatter(ob, [idxn, _splat16(fc)], hb)

            pltpu.sync_copy(oa.at[pl.ds(0, ch)], ua_hbm.at[pl.ds(n0, ch)])
            pltpu.sync_copy(ob.at[pl.ds(0, ch)], ub_hbm.at[pl.ds(n0, ch)])
            pltpu.sync_copy(od.at[pl.ds(0, ch)], db_hbm.at[pl.ds(n0, ch)])

        wid = 2 * s + c
        base = wid * (4 * CH)

        @pl.loop(0, 4)
        def _(kc):
            run_chunk(base + kc * CH, CH)

        @pl.when(c == 0)
        def _():
            run_chunk(32 * 4 * CH + s * TAIL, TAIL)

    return k


def _make_row_prop_kernel(E, NP, N, D):
    GROUPS = E // EPG
    NIT = 198
    HALF = N // 2
    zrows = 3128
    arows = 16 * zrows
    w0 = 3080

    scratch = []
    for _ in range(3):
        scratch += [
            pltpu.VMEM((KSUB, 128), I32),
            pltpu.VMEM((KSUB, 128), I32),
            pltpu.VMEM((KSUB, 128), I32),
            pltpu.VMEM((EPG, D), F32),
            pltpu.SemaphoreType.DMA,
            pltpu.SemaphoreType.DMA,
            pltpu.SemaphoreType.DMA,
        ]
    scratch += [
        pltpu.VMEM((zrows, D), F32),
        pltpu.VMEM_SHARED((arows, D), F32),
    ]

    @functools.partial(
        pl.kernel,
        out_type=(jax.ShapeDtypeStruct((NP, D), F32),
                  jax.ShapeDtypeStruct((NP, D), F32)),
        mesh=_sc_mesh(),
        compiler_params=_SC_PARAMS,
        scratch_types=scratch,
    )
    def k(ei_hbm, u1a_hbm, u1b_hbm, zeros_hbm, ta_hbm, tb_hbm, *refs):
        sets = tuple(refs[7 * m:7 * m + 7] for m in range(3))
        bbuf, acc = refs[21], refs[22]
        c = lax.axis_index("c")
        s = lax.axis_index("s")
        base = c * HALF
        garb = HALF + s

        def g_of(ii):
            return s + 16 * ii

        def valid(g):
            return (g >= 0) & (g < GROUPS)

        def fire_edges(g, st):
            pltpu.async_copy(ei_hbm.at[0, pl.ds(g * KSUB, KSUB)],
                             st[0], st[4])
            pltpu.async_copy(ei_hbm.at[1, pl.ds(g * KSUB, KSUB)],
                             st[1], st[4])

        def wait_edges(g, st):
            pltpu.make_async_copy(ei_hbm.at[0, pl.ds(g * KSUB, KSUB)],
                                  st[0], st[4]).wait()
            pltpu.make_async_copy(ei_hbm.at[1, pl.ds(g * KSUB, KSUB)],
                                  st[1], st[4]).wait()

        for u_hbm, o_hbm in ((u1a_hbm, ta_hbm), (u1b_hbm, tb_hbm)):
            pltpu.sync_copy(zeros_hbm, acc.at[pl.ds(s * zrows, zrows)])
            plsc.subcore_barrier()

            def fire_gathers(st):
                for kk in range(KSUB):
                    pltpu.async_copy(u_hbm.at[st[0].at[kk]],
                                     st[3].at[pl.ds(kk * 128, 128)], st[5])

            def drain_gathers(st):
                for kk in range(KSUB):
                    pltpu.make_async_copy(
                        u_hbm.at[st[0].at[kk]],
                        st[3].at[pl.ds(kk * 128, 128)], st[5]).wait()

            def fire_scatters(st):
                for kk in range(KSUB):
                    pltpu.async_copy(st[3].at[pl.ds(kk * 128, 128)],
                                     acc.at[st[2].at[kk]], st[6], add=True)

            def drain_scatters(st):
                for kk in range(KSUB):
                    pltpu.make_async_copy(
                        st[3].at[pl.ds(kk * 128, 128)],
                        acc.at[st[2].at[kk]], st[6]).wait()

            def do_slot(ii, m):
                st = sets[m]
                stp = sets[(m + 2) % 3]
                db, lb = st[1], st[2]
                g = g_of(ii)
                gp = g_of(ii - 1)

                @pl.when(valid(g))
                def _():
                    wait_edges(g, st)

                    @pl.when(ii >= 3)
                    def _():
                        drain_scatters(st)

                    fire_gathers(st)
                    for kk in range(KSUB):
                        for j in range(8):
                            sl = pl.ds(j * 16, 16)
                            loc = db[kk, sl] - base
                            ok = (loc >= 0) & (loc < HALF)
                            lb[kk, sl] = jnp.where(ok, loc, garb)

                @pl.when(valid(gp))
                def _():
                    drain_gathers(stp)
                    fire_scatters(stp)

                    @pl.when(valid(g_of(ii + 2)))
                    def _():
                        fire_edges(g_of(ii + 2), stp)

            for m in (0, 1, 2):
                @pl.when(valid(g_of(m)))
                def _(m=m):
                    fire_edges(g_of(m), sets[m])

            @pl.loop(0, NIT, step=3)
            def _(i):
                do_slot(i, 0)
                do_slot(i + 1, 1)
                do_slot(i + 2, 2)

            for ii in range(NIT - 6, NIT):
                @pl.when(valid(g_of(ii)) & ~valid(g_of(ii + 3)))
                def _(ii=ii):
                    drain_scatters(sets[ii % 3])

            plsc.subcore_barrier()
            pltpu.sync_copy(acc.at[pl.ds(s * zrows, w0)],
                            bbuf.at[pl.ds(0, w0)])
            pltpu.sync_copy(bbuf.at[pl.ds(0, w0)],
                            o_hbm.at[pl.ds(base + s * zrows, w0)])

            @pl.when(s < 15)
            def _():
                pltpu.sync_copy(acc.at[pl.ds(s * zrows + w0, zrows - w0)],
                                bbuf.at[pl.ds(w0, zrows - w0)])
                pltpu.sync_copy(bbuf.at[pl.ds(w0, zrows - w0)],
                                o_hbm.at[pl.ds(base + s * zrows + w0,
                                               zrows - w0)])

            plsc.subcore_barrier()

    return k


def _make_pool_kernel(NP, G):
    slc = NP // 16
    nk = slc // 128

    @functools.partial(
        pl.kernel,
        out_type=jax.ShapeDtypeStruct((G,), F32),
        mesh=_sc_mesh(),
        compiler_params=_SC_PARAMS,
        scratch_types=[
            pltpu.VMEM((slc,), F32),
            pltpu.VMEM((nk, 128), I32),
            pltpu.VMEM((128,), F32),
            pltpu.VMEM((2 * G,), F32),
            pltpu.VMEM((16,), F32),
            pltpu.VMEM_SHARED((G + 128,), F32),
            pltpu.VMEM_SHARED((G + 128,), F32),
            pltpu.SemaphoreType.DMA,
        ],
    )
    def k(v_hbm, bat_hbm, bl_hbm, zeros_hbm, out_hbm,
          vbuf, bbuf, obuf, stg, blv, sums, cnts, sem):
        c = lax.axis_index("c")
        s = lax.axis_index("s")

        @pl.when(c == 0)
        def _():
            @pl.when(s == 0)
            def _():
                pltpu.sync_copy(zeros_hbm.at[pl.ds(0, G + 128)], sums)
                pltpu.sync_copy(zeros_hbm.at[pl.ds(0, G + 128)], cnts)
            for j in range(8):
                obuf[pl.ds(j * 16, 16)] = jnp.full((16,), 1.0, F32)
            pltpu.sync_copy(v_hbm.at[pl.ds(s * slc, slc)], vbuf)
            pltpu.sync_copy(bat_hbm.at[pl.ds(s * nk, nk)], bbuf)
            plsc.subcore_barrier()
            for kk in range(nk):
                pltpu.async_copy(vbuf.at[pl.ds(kk * 128, 128)],
                                 sums.at[bbuf.at[kk]], sem, add=True)
                pltpu.async_copy(obuf, cnts.at[bbuf.at[kk]], sem, add=True)
            for kk in range(nk):
                pltpu.make_async_copy(vbuf.at[pl.ds(kk * 128, 128)],
                                      sums.at[bbuf.at[kk]], sem).wait()
                pltpu.make_async_copy(obuf, cnts.at[bbuf.at[kk]],
                                      sem).wait()
            plsc.subcore_barrier()

            @pl.when(s == 0)
            def _():
                pltpu.sync_copy(bl_hbm, blv)
                pltpu.sync_copy(sums.at[pl.ds(0, G)], stg.at[pl.ds(0, G)])
                pltpu.sync_copy(cnts.at[pl.ds(0, G)], stg.at[pl.ds(G, G)])
                bl16 = plsc.load_gather(blv, [_splat16(0)])

                @pl.loop(0, G // 16)
                def _(i):
                    sl = pl.ds(i * 16, 16)
                    sv = stg[sl]
                    cv = stg[pl.ds(G + i * 16, 16)]
                    vbuf[sl] = sv / jnp.maximum(cv, 1.0) + bl16

                pltpu.sync_copy(vbuf.at[pl.ds(0, G)], out_hbm)

    return k


def _dinv_u0_body(d0, d1, x, dinv_o, u0_o):
    deg = d0[...] + d1[...] + 1.0
    dinv = lax.rsqrt(deg)
    dinv_o[...] = dinv
    u0_o[...] = dinv * x[...]


def _layer2_body(ta, tb, ua, ub, dv, w2a, w2b, b2e, wle, v_o):
    hi = lax.Precision.HIGHEST
    qa = dv[...] * (ta[...] + ua[...])
    qb = dv[...] * (tb[...] + ub[...])
    h2 = jnp.maximum(
        jnp.dot(qa, w2a[...], preferred_element_type=F32, precision=hi)
        + jnp.dot(qb, w2b[...], preferred_element_type=F32, precision=hi)
        + b2e[...], 0.0)
    v_o[...] = jnp.dot(h2, wle[...], preferred_element_type=F32,
                       precision=hi)


def kernel(x, edge_index, batch, W1, b1, W2, b2, Wl, bl):
    N = x.shape[0]
    E = edge_index.shape[1]
    G = 256
    NP = 100352
    D = 16
    NF = NP * D // 128

    ei3 = edge_index.reshape(2, E // 128, 128)
    xp = jnp.pad(x[:, 0], (0, NP - N))
    batp2 = jnp.pad(batch, (0, NP - N),
                    constant_values=G).reshape(NP // 128, 128)
    zeros1 = jnp.zeros((NP // 16,), F32)
    zeros2 = jnp.zeros((3128, D), F32)
    eye8 = jnp.eye(8, dtype=F32)
    w2a = jnp.kron(eye8, W2[:16, :])
    w2b = jnp.kron(eye8, W2[16:, :])
    b2e = jnp.tile(b2, 8).reshape(1, 512)
    wle = jnp.kron(eye8, Wl)
    blp = jnp.pad(bl, (0, 15))

    degp = _make_deg_scalar_kernel(E, NP, gather=False)(ei3, zeros1, zeros1)

    dinv2, u02 = pl.pallas_call(
        _dinv_u0_body,
        out_shape=[jax.ShapeDtypeStruct((784, 128), F32)] * 2,
    )(degp[:NP].reshape(784, 128), degp[NP:].reshape(784, 128),
      xp.reshape(784, 128))
    dinvf = dinv2.reshape(NP)
    u0f = u02.reshape(NP)

    tp = _make_deg_scalar_kernel(E, NP, gather=True)(ei3, u0f, zeros1)

    u1a, u1b, dvb = _make_u1_kernel(NP)(tp, u0f, dinvf, W1.reshape(32), b1)

    t2a, t2b = _make_row_prop_kernel(E, NP, N, D)(ei3, u1a, u1b, zeros2)

    fl = pl.BlockSpec((256, 128), lambda i: (i, 0))
    cst = pl.BlockSpec((128, 512), lambda i: (0, 0))
    v = pl.pallas_call(
        _layer2_body,
        grid=(NF // 256,),
        in_specs=[fl, fl, fl, fl, fl, cst, cst,
                  pl.BlockSpec((1, 512), lambda i: (0, 0)),
                  pl.BlockSpec((512, 8), lambda i: (0, 0))],
        out_specs=pl.BlockSpec((256, 8), lambda i: (i, 0)),
        out_shape=jax.ShapeDtypeStruct((NF, 8), F32),
    )(t2a.reshape(NF, 128), t2b.reshape(NF, 128), u1a.reshape(NF, 128),
      u1b.reshape(NF, 128), dvb.reshape(NF, 128), w2a, w2b, b2e, wle)

    out = _make_pool_kernel(NP, G)(v.reshape(NP), batp2, blp, zeros1)
    return out.reshape(G, 1)


# speedup vs baseline: 74.2326x; 1.0735x over previous
import functools

import jax
import jax.numpy as jnp
from jax import lax
from jax.experimental import pallas as pl
from jax.experimental.pallas import tpu as pltpu
from jax.experimental.pallas import tpu_sc as plsc

F32 = jnp.float32
I32 = jnp.int32

EPG = 512
KSUB = EPG // 128


def _sc_mesh():
    return plsc.VectorSubcoreMesh(core_axis_name="c", subcore_axis_name="s")


_SC_PARAMS = pltpu.CompilerParams(needs_layout_passes=False,
                                  use_tc_tiling_on_sc=False)


def _splat16(v):
    return jnp.full((16,), v, I32)


def _make_deg_scalar_kernel(E, NP, gather):
    GROUPS = E // EPG
    PER_C = GROUPS // 2
    NIT = 98
    slc = NP // 16

    scratch = []
    for _ in range(2):
        scratch += [
            pltpu.VMEM((KSUB, 128), I32),
            pltpu.VMEM((KSUB, 128), I32),
            pltpu.VMEM((KSUB, 128), I32),
            pltpu.VMEM((KSUB, 128), F32),
            pltpu.SemaphoreType.DMA,
            pltpu.SemaphoreType.DMA,
        ]
    scratch += [
        pltpu.VMEM((slc,), F32),
        pltpu.VMEM((NP,), F32),
        pltpu.VMEM_SHARED((NP,), F32),
    ]

    @functools.partial(
        pl.kernel,
        out_type=jax.ShapeDtypeStruct((2 * NP,), F32),
        mesh=_sc_mesh(),
        compiler_params=_SC_PARAMS,
        scratch_types=scratch,
    )
    def k(ei_hbm, u0_hbm, zeros_hbm, out_hbm, *refs):
        sets = tuple(refs[6 * m:6 * m + 6] for m in range(2))
        bbuf, u0v, acc = refs[12], refs[13], refs[14]
        c = lax.axis_index("c")
        s = lax.axis_index("s")
        start = c * PER_C
        cnt = PER_C + c
        pltpu.sync_copy(zeros_hbm, acc.at[pl.ds(s * slc, slc)])
        if gather:
            pltpu.sync_copy(u0_hbm, u0v)
        plsc.subcore_barrier()

        def g_of(ii):
            return start + s + 16 * ii

        def valid(g):
            return g < start + cnt

        def fire_edges(g, st):
            if gather:
                pltpu.async_copy(ei_hbm.at[0, pl.ds(g * KSUB, KSUB)],
                                 st[0], st[4])
            pltpu.async_copy(ei_hbm.at[1, pl.ds(g * KSUB, KSUB)],
                             st[1], st[4])

        def wait_edges(g, st):
            if gather:
                pltpu.make_async_copy(ei_hbm.at[0, pl.ds(g * KSUB, KSUB)],
                                      st[0], st[4]).wait()
            pltpu.make_async_copy(ei_hbm.at[1, pl.ds(g * KSUB, KSUB)],
                                  st[1], st[4]).wait()

        def drain_scatters(st):
            for kk in range(KSUB):
                pltpu.make_async_copy(
                    st[3].at[kk], acc.at[st[2].at[kk]], st[5]).wait()

        def do_slot(ii, b):
            st = sets[b]
            sb, db, ib, vb, es, ss = st
            g = g_of(ii)

            @pl.when(valid(g))
            def _():
                wait_edges(g, st)

                @pl.when(ii >= 2)
                def _():
                    drain_scatters(st)

                for kk in range(KSUB):
                    for j in range(8):
                        sl = pl.ds(j * 16, 16)
                        ib[kk, sl] = db[kk, sl]
                        if gather:
                            vb[kk, sl] = plsc.load_gather(u0v, [sb[kk, sl]])
                        else:
                            vb[kk, sl] = jnp.full((16,), 1.0, F32)
                for kk in range(KSUB):
                    pltpu.async_copy(vb.at[kk], acc.at[ib.at[kk]], ss,
                                     add=True)

                @pl.when(valid(g_of(ii + 2)))
                def _():
                    fire_edges(g_of(ii + 2), st)

        for b in (0, 1):
            @pl.when(valid(g_of(b)))
            def _(b=b):
                fire_edges(g_of(b), sets[b])

        @pl.loop(0, NIT, step=2)
        def _(i):
            do_slot(i, 0)
            do_slot(i + 1, 1)

        for ii in (NIT - 3, NIT - 2, NIT - 1):
            @pl.when(valid(g_of(ii)) & ~valid(g_of(ii + 2)))
            def _(ii=ii):
                drain_scatters(sets[ii % 2])

        plsc.subcore_barrier()
        pltpu.sync_copy(acc.at[pl.ds(s * slc, slc)], bbuf)
        pltpu.sync_copy(bbuf, out_hbm.at[pl.ds(c * NP + s * slc, slc)])

    return k


def _make_u1_kernel(NP):
    CH = 768
    TAIL = 128

    @functools.partial(
        pl.kernel,
        out_type=(jax.ShapeDtypeStruct((NP, 16), F32),
                  jax.ShapeDtypeStruct((NP, 16), F32),
                  jax.ShapeDtypeStruct((NP, 16), F32)),
        mesh=_sc_mesh(),
        compiler_params=_SC_PARAMS,
        scratch_types=[
            pltpu.VMEM((CH,), F32),
            pltpu.VMEM((CH,), F32),
            pltpu.VMEM((CH,), F32),
            pltpu.VMEM((CH,), F32),
            pltpu.VMEM((CH,), F32),
            pltpu.VMEM((CH, 16), F32),
            pltpu.VMEM((CH, 16), F32),
            pltpu.VMEM((CH, 16), F32),
            pltpu.VMEM((48,), F32),
            pltpu.VMEM((48,), F32),
        ],
    )
    def k(tp_hbm, u0_hbm, dv_hbm, w1_hbm, b1_hbm, ua_hbm, ub_hbm, db_hbm,
          t0b, t1b, u0b, dvs, pb, oa, ob, od, w1v, b1v):
        c = lax.axis_index("c")
        s = lax.axis_index("s")
        pltpu.sync_copy(w1_hbm, w1v)
        pltpu.sync_copy(b1_hbm, b1v)
        iota16 = lax.iota(I32, 16)

        def run_chunk(n0, ch):
            ng = ch // 16
            pltpu.sync_copy(tp_hbm.at[pl.ds(n0, ch)], t0b.at[pl.ds(0, ch)])
            pltpu.sync_copy(tp_hbm.at[pl.ds(NP + n0, ch)],
                            t1b.at[pl.ds(0, ch)])
            pltpu.sync_copy(u0_hbm.at[pl.ds(n0, ch)], u0b.at[pl.ds(0, ch)])
            pltpu.sync_copy(dv_hbm.at[pl.ds(n0, ch)], dvs.at[pl.ds(0, ch)])

            @pl.loop(0, ng)
            def _(g):
                sl = pl.ds(g * 16, 16)
                dv = dvs[sl]
                pb[sl] = dv * (t0b[sl] + t1b[sl] + u0b[sl])
                idxn = g * 16 + iota16
                for fc in range(16):
                    plsc.store_scatter(od, [idxn, _splat16(fc)], dv)

            for half, obuf in ((0, oa), (1, ob)):
                for fc in range(16):
                    wf = plsc.load_gather(w1v,
                                          [_splat16(16 + half * 16 + fc)])
                    bf = plsc.load_gather(b1v,
                                          [_splat16(16 + half * 16 + fc)])

                    @pl.loop(0, ng)
                    def _(g):
                        sl = pl.ds(g * 16, 16)
                        idxn = g * 16 + iota16
                        h = jnp.maximum(pb[sl] * wf + bf, 0.0) * dvs[sl]
                        plsc.store_scatter(obuf, [idxn, _splat16(fc)], h)

            pltpu.sync_copy(oa.at[pl.ds(0, ch)], ua_hbm.at[pl.ds(n0, ch)])
            pltpu.sync_copy(ob.at[pl.ds(0, ch)], ub_hbm.at[pl.ds(n0, ch)])
            pltpu.sync_copy(od.at[pl.ds(0, ch)], db_hbm.at[pl.ds(n0, ch)])

        wid = 2 * s + c
        base = wid * (4 * CH)

        @pl.loop(0, 4)
        def _(kc):
            run_chunk(base + kc * CH, CH)

        @pl.when(c == 0)
        def _():
            run_chunk(32 * 4 * CH + s * TAIL, TAIL)

    return k


def _make_row_prop_kernel(E, NP, N, D):
    GROUPS = E // EPG
    NIT = 198
    HALF = N // 2
    zrows = 3128
    arows = 16 * zrows
    w0 = 3080

    scratch = []
    for _ in range(3):
        scratch += [
            pltpu.VMEM((KSUB, 128), I32),
            pltpu.VMEM((KSUB, 128), I32),
            pltpu.VMEM((KSUB, 128), I32),
            pltpu.VMEM((EPG, D), F32),
            pltpu.SemaphoreType.DMA,
            pltpu.SemaphoreType.DMA,
            pltpu.SemaphoreType.DMA,
        ]
    scratch += [
        pltpu.VMEM((zrows, D), F32),
        pltpu.VMEM_SHARED((arows, D), F32),
    ]

    @functools.partial(
        pl.kernel,
        out_type=(jax.ShapeDtypeStruct((NP, D), F32),
                  jax.ShapeDtypeStruct((NP, D), F32)),
        mesh=_sc_mesh(),
        compiler_params=_SC_PARAMS,
        scratch_types=scratch,
    )
    def k(ei_hbm, u1a_hbm, u1b_hbm, zeros_hbm, ta_hbm, tb_hbm, *refs):
        sets = tuple(refs[7 * m:7 * m + 7] for m in range(3))
        bbuf, acc = refs[21], refs[22]
        c = lax.axis_index("c")
        s = lax.axis_index("s")
        base = c * HALF
        garb = HALF + s

        def g_of(ii):
            return s + 16 * ii

        def valid(g):
            return (g >= 0) & (g < GROUPS)

        def fire_edges(g, st):
            pltpu.async_copy(ei_hbm.at[0, pl.ds(g * KSUB, KSUB)],
                             st[0], st[4])
            pltpu.async_copy(ei_hbm.at[1, pl.ds(g * KSUB, KSUB)],
                             st[1], st[4])

        def wait_edges(g, st):
            pltpu.make_async_copy(ei_hbm.at[0, pl.ds(g * KSUB, KSUB)],
                                  st[0], st[4]).wait()
            pltpu.make_async_copy(ei_hbm.at[1, pl.ds(g * KSUB, KSUB)],
                                  st[1], st[4]).wait()

        for u_hbm, o_hbm in ((u1a_hbm, ta_hbm), (u1b_hbm, tb_hbm)):
            pltpu.sync_copy(zeros_hbm, acc.at[pl.ds(s * zrows, zrows)])
            plsc.subcore_barrier()

            def fire_gathers(st):
                for kk in range(KSUB):
                    pltpu.async_copy(u_hbm.at[st[0].at[kk]],
                                     st[3].at[pl.ds(kk * 128, 128)], st[5])

            def drain_gathers(st):
                for kk in range(KSUB):
                    pltpu.make_async_copy(
                        u_hbm.at[st[0].at[kk]],
                        st[3].at[pl.ds(kk * 128, 128)], st[5]).wait()

            def fire_scatters(st):
                for kk in range(KSUB):
                    pltpu.async_copy(st[3].at[pl.ds(kk * 128, 128)],
                                     acc.at[st[2].at[kk]], st[6], add=True)

            def drain_scatters(st):
                for kk in range(KSUB):
                    pltpu.make_async_copy(
                        st[3].at[pl.ds(kk * 128, 128)],
                        acc.at[st[2].at[kk]], st[6]).wait()

            def do_slot(ii, m):
                st = sets[m]
                stp = sets[(m + 2) % 3]
                db, lb = st[1], st[2]
                g = g_of(ii)
                gp = g_of(ii - 1)

                @pl.when(valid(g))
                def _():
                    wait_edges(g, st)

                    @pl.when(ii >= 3)
                    def _():
                        drain_scatters(st)

                    fire_gathers(st)
                    for kk in range(KSUB):
                        for j in range(8):
                            sl = pl.ds(j * 16, 16)
                            loc = db[kk, sl] - base
                            ok = (loc >= 0) & (loc < HALF)
                            lb[kk, sl] = jnp.where(ok, loc, garb)

                @pl.when(valid(gp))
                def _():
                    drain_gathers(stp)
                    fire_scatters(stp)

                    @pl.when(valid(g_of(ii + 2)))
                    def _():
                        fire_edges(g_of(ii + 2), stp)

            for m in (0, 1, 2):
                @pl.when(valid(g_of(m)))
                def _(m=m):
                    fire_edges(g_of(m), sets[m])

            @pl.loop(0, NIT, step=3)
            def _(i):
                do_slot(i, 0)
                do_slot(i + 1, 1)
                do_slot(i + 2, 2)

            for ii in range(NIT - 6, NIT):
                @pl.when(valid(g_of(ii)) & ~valid(g_of(ii + 3)))
                def _(ii=ii):
                    drain_scatters(sets[ii % 3])

            plsc.subcore_barrier()
            pltpu.sync_copy(acc.at[pl.ds(s * zrows, w0)],
                            bbuf.at[pl.ds(0, w0)])
            pltpu.sync_copy(bbuf.at[pl.ds(0, w0)],
                            o_hbm.at[pl.ds(base + s * zrows, w0)])

            @pl.when(s < 15)
            def _():
                pltpu.sync_copy(acc.at[pl.ds(s * zrows + w0, zrows - w0)],
                                bbuf.at[pl.ds(w0, zrows - w0)])
                pltpu.sync_copy(bbuf.at[pl.ds(w0, zrows - w0)],
                                o_hbm.at[pl.ds(base + s * zrows + w0,
                                               zrows - w0)])

            plsc.subcore_barrier()

    return k


def _make_pool_kernel(NP, G):
    slc = NP // 16
    nk = slc // 128

    @functools.partial(
        pl.kernel,
        out_type=jax.ShapeDtypeStruct((G,), F32),
        mesh=_sc_mesh(),
        compiler_params=_SC_PARAMS,
        scratch_types=[
            pltpu.VMEM((slc,), F32),
            pltpu.VMEM((nk, 128), I32),
            pltpu.VMEM((128,), F32),
            pltpu.VMEM((2 * G,), F32),
            pltpu.VMEM((32,), F32),
            pltpu.VMEM_SHARED((G + 128,), F32),
            pltpu.VMEM_SHARED((G + 128,), F32),
            pltpu.SemaphoreType.DMA,
        ],
    )
    def k(v_hbm, bat_hbm, bl_hbm, zeros_hbm, out_hbm,
          vbuf, bbuf, obuf, stg, blv, sums, cnts, sem):
        c = lax.axis_index("c")
        s = lax.axis_index("s")

        @pl.when(c == 0)
        def _():
            @pl.when(s == 0)
            def _():
                pltpu.sync_copy(zeros_hbm.at[pl.ds(0, G + 128)], sums)
                pltpu.sync_copy(zeros_hbm.at[pl.ds(0, G + 128)], cnts)
            for j in range(8):
                obuf[pl.ds(j * 16, 16)] = jnp.full((16,), 1.0, F32)
            pltpu.sync_copy(v_hbm.at[pl.ds(s * slc, slc)], vbuf)
            pltpu.sync_copy(bat_hbm.at[pl.ds(s * nk, nk)], bbuf)
            plsc.subcore_barrier()
            for kk in range(nk):
                pltpu.async_copy(vbuf.at[pl.ds(kk * 128, 128)],
                                 sums.at[bbuf.at[kk]], sem, add=True)
                pltpu.async_copy(obuf, cnts.at[bbuf.at[kk]], sem, add=True)
            for kk in range(nk):
                pltpu.make_async_copy(vbuf.at[pl.ds(kk * 128, 128)],
                                      sums.at[bbuf.at[kk]], sem).wait()
                pltpu.make_async_copy(obuf, cnts.at[bbuf.at[kk]],
                                      sem).wait()
            plsc.subcore_barrier()

            @pl.when(s == 0)
            def _():
                pltpu.sync_copy(bl_hbm, blv)
                pltpu.sync_copy(sums.at[pl.ds(0, G)], stg.at[pl.ds(0, G)])
                pltpu.sync_copy(cnts.at[pl.ds(0, G)], stg.at[pl.ds(G, G)])
                bl16 = plsc.load_gather(blv, [_splat16(16)])

                @pl.loop(0, G // 16)
                def _(i):
                    sl = pl.ds(i * 16, 16)
                    sv = stg[sl]
                    cv = stg[pl.ds(G + i * 16, 16)]
                    vbuf[sl] = sv / jnp.maximum(cv, 1.0) + bl16

                pltpu.sync_copy(vbuf.at[pl.ds(0, G)], out_hbm)

    return k


def _dinv_u0_body(d0, d1, x, dinv_o, u0_o):
    deg = d0[...] + d1[...] + 1.0
    dinv = lax.rsqrt(deg)
    dinv_o[...] = dinv
    u0_o[...] = dinv * x[...]


def _layer2_body(ta, tb, ua, ub, dv, w2a, w2b, b2e, wle, v_o):
    qa = dv[...] * (ta[...] + ua[...])
    qb = dv[...] * (tb[...] + ub[...])
    h2 = jnp.maximum(
        jnp.dot(qa, w2a[...], preferred_element_type=F32)
        + jnp.dot(qb, w2b[...], preferred_element_type=F32)
        + b2e[...], 0.0)
    v_o[...] = jnp.dot(h2, wle[...], preferred_element_type=F32)


def kernel(x, edge_index, batch, W1, b1, W2, b2, Wl, bl):
    N = x.shape[0]
    E = edge_index.shape[1]
    G = 256
    NP = 100352
    D = 16
    NF = NP * D // 128

    ei3 = edge_index.reshape(2, E // 128, 128)
    xp = jnp.pad(x[:, 0], (0, NP - N))
    batp2 = jnp.pad(batch, (0, NP - N),
                    constant_values=G).reshape(NP // 128, 128)
    zeros1 = jnp.zeros((NP // 16,), F32)
    zeros2 = jnp.zeros((3128, D), F32)
    eye8 = jnp.eye(8, dtype=F32)
    w2a = jnp.kron(eye8, W2[:16, :])
    w2b = jnp.kron(eye8, W2[16:, :])
    b2e = jnp.tile(b2, 8).reshape(1, 512)
    wle = jnp.kron(eye8, Wl)
    blp = jnp.pad(bl, (16, 15))

    degp = _make_deg_scalar_kernel(E, NP, gather=False)(ei3, zeros1, zeros1)

    dinv2, u02 = pl.pallas_call(
        _dinv_u0_body,
        out_shape=[jax.ShapeDtypeStruct((784, 128), F32)] * 2,
    )(degp[:NP].reshape(784, 128), degp[NP:].reshape(784, 128),
      xp.reshape(784, 128))
    dinvf = dinv2.reshape(NP)
    u0f = u02.reshape(NP)

    tp = _make_deg_scalar_kernel(E, NP, gather=True)(ei3, u0f, zeros1)

    w1p = jnp.pad(W1.reshape(32), (16, 0))
    b1p = jnp.pad(b1, (16, 0))
    u1a, u1b, dvb = _make_u1_kernel(NP)(tp, u0f, dinvf, w1p, b1p)

    t2a, t2b = _make_row_prop_kernel(E, NP, N, D)(ei3, u1a, u1b, zeros2)

    fl = pl.BlockSpec((256, 128), lambda i: (i, 0))
    cst = pl.BlockSpec((128, 512), lambda i: (0, 0))
    v = pl.pallas_call(
        _layer2_body,
        grid=(NF // 256,),
        in_specs=[fl, fl, fl, fl, fl, cst, cst,
                  pl.BlockSpec((1, 512), lambda i: (0, 0)),
                  pl.BlockSpec((512, 8), lambda i: (0, 0))],
        out_specs=pl.BlockSpec((256, 8), lambda i: (i, 0)),
        out_shape=jax.ShapeDtypeStruct((NF, 8), F32),
    )(t2a.reshape(NF, 128), t2b.reshape(NF, 128), u1a.reshape(NF, 128),
      u1b.reshape(NF, 128), dvb.reshape(NF, 128), w2a, w2b, b2e, wle)

    out = _make_pool_kernel(NP, G)(v.reshape(NP), batp2, blp, zeros1)
    return out.reshape(G, 1)
